# baseline (speedup 1.0000x reference)
import functools

import jax
import jax.numpy as jnp
from jax import lax
from jax.experimental import pallas as pl
from jax.experimental.pallas import tpu as pltpu
from jax.experimental.pallas import tpu_sc as plsc

_LANES = 16
_NW = 32


def _hat_mask_body(emb_hbm, idx_hbm, s_hbm, out_hbm, idx_v, s_v, row_v,
                   out_v, sem_i, sem_s, sem_g):
    slc = out_v.shape[0]
    wid = lax.axis_index("s") * 2 + lax.axis_index("c")
    cp_i = pltpu.async_copy(idx_hbm.at[wid], idx_v, sem_i)
    cp_s = pltpu.async_copy(s_hbm, s_v, sem_s)
    cp_i.wait()
    cp_g = pltpu.async_copy(emb_hbm.at[idx_v], row_v, sem_g)
    cp_s.wait()
    cp_g.wait()
    sv = s_v[...]
    for j in range(slc // _LANES):
        x = row_v[0, pl.ds(j * _LANES, _LANES)]
        out_v[pl.ds(j * _LANES, _LANES)] = 1.0 / (1.0 + jnp.exp(-(sv * x)))
    pltpu.sync_copy(out_v, out_hbm.at[pl.ds(wid * slc, slc)])


def kernel(embeddings, task_id, s):
    n_tasks, n_units = embeddings.shape
    slc = n_units // _NW
    emb2 = embeddings.reshape(n_tasks * _NW, slc)
    idx = (jnp.int32(task_id) * _NW
           + jnp.arange(_NW, dtype=jnp.int32)).reshape(_NW, 1)
    s_vec = jnp.full((_LANES,), s, dtype=jnp.float32)

    f = functools.partial(
        pl.kernel,
        out_type=jax.ShapeDtypeStruct((n_units,), jnp.float32),
        mesh=plsc.VectorSubcoreMesh(core_axis_name="c", subcore_axis_name="s"),
        scratch_types=[
            pltpu.VMEM((1,), jnp.int32),
            pltpu.VMEM((_LANES,), jnp.float32),
            pltpu.VMEM((1, slc), jnp.float32),
            pltpu.VMEM((slc,), jnp.float32),
            pltpu.SemaphoreType.DMA,
            pltpu.SemaphoreType.DMA,
            pltpu.SemaphoreType.DMA,
        ],
    )(_hat_mask_body)
    return f(emb2, idx, s_vec)


# speedup vs baseline: 1.0854x; 1.0854x over previous
import functools

import jax
import jax.numpy as jnp
from jax import lax
from jax.experimental import pallas as pl
from jax.experimental.pallas import tpu as pltpu
from jax.experimental.pallas import tpu_sc as plsc

_LANES = 16
_NW = 16


def _hat_mask_body(emb_hbm, idx_hbm, s_hbm, out_hbm, idx_v, s_v, row_v,
                   out_v, sem_i, sem_s, sem_g):
    slc = out_v.shape[0]
    wid = lax.axis_index("s")
    cp_i = pltpu.async_copy(idx_hbm.at[wid], idx_v, sem_i)
    cp_s = pltpu.async_copy(s_hbm, s_v, sem_s)
    cp_i.wait()
    cp_g = pltpu.async_copy(emb_hbm.at[idx_v], row_v, sem_g)
    cp_s.wait()
    cp_g.wait()
    sv = s_v[...]
    for j in range(slc // _LANES):
        x = row_v[0, pl.ds(j * _LANES, _LANES)]
        out_v[pl.ds(j * _LANES, _LANES)] = 1.0 / (1.0 + jnp.exp(-(sv * x)))
    pltpu.sync_copy(out_v, out_hbm.at[pl.ds(wid * slc, slc)])


def kernel(embeddings, task_id, s):
    n_tasks, n_units = embeddings.shape
    slc = n_units // _NW
    emb2 = embeddings.reshape(n_tasks * _NW, slc)
    idx = (jnp.int32(task_id) * _NW
           + jnp.arange(_NW, dtype=jnp.int32)).reshape(_NW, 1)
    s_vec = jnp.full((_LANES,), s, dtype=jnp.float32)

    f = functools.partial(
        pl.kernel,
        out_type=jax.ShapeDtypeStruct((n_units,), jnp.float32),
        mesh=plsc.VectorSubcoreMesh(core_axis_name="c", subcore_axis_name="s",
                                    num_cores=1),
        scratch_types=[
            pltpu.VMEM((1,), jnp.int32),
            pltpu.VMEM((_LANES,), jnp.float32),
            pltpu.VMEM((1, slc), jnp.float32),
            pltpu.VMEM((slc,), jnp.float32),
            pltpu.SemaphoreType.DMA,
            pltpu.SemaphoreType.DMA,
            pltpu.SemaphoreType.DMA,
        ],
    )(_hat_mask_body)
    return f(emb2, idx, s_vec)


# speedup vs baseline: 1.1311x; 1.0420x over previous
import functools

import jax
import jax.numpy as jnp
from jax import lax
from jax.experimental import pallas as pl
from jax.experimental.pallas import tpu as pltpu
from jax.experimental.pallas import tpu_sc as plsc

_LANES = 16
_NW = 16


def _hat_mask_body(emb_hbm, idx_hbm, s_hbm, out_hbm, idx_v, s_v, row_v,
                   out_v, sem_i, sem_s, sem_g):
    slc = out_v.shape[0]
    wid = lax.axis_index("s")
    cp_i = pltpu.async_copy(idx_hbm, idx_v, sem_i)
    cp_s = pltpu.async_copy(s_hbm, s_v, sem_s)
    cp_i.wait()
    cp_g = pltpu.async_copy(emb_hbm.at[idx_v], row_v, sem_g)
    cp_s.wait()
    cp_g.wait()
    sv = s_v[...]
    base = wid * slc
    for j in range(slc // _LANES):
        x = row_v[0, pl.ds(base + j * _LANES, _LANES)]
        out_v[pl.ds(j * _LANES, _LANES)] = 1.0 / (1.0 + jnp.exp(-(sv * x)))
    pltpu.sync_copy(out_v, out_hbm.at[pl.ds(base, slc)])


def kernel(embeddings, task_id, s):
    n_tasks, n_units = embeddings.shape
    slc = n_units // _NW
    idx = jnp.reshape(jnp.int32(task_id), (1,))
    s_vec = jnp.full((_LANES,), s, dtype=jnp.float32)

    f = functools.partial(
        pl.kernel,
        out_type=jax.ShapeDtypeStruct((n_units,), jnp.float32),
        mesh=plsc.VectorSubcoreMesh(core_axis_name="c", subcore_axis_name="s",
                                    num_cores=1),
        compiler_params=pltpu.CompilerParams(use_tc_tiling_on_sc=True),
        scratch_types=[
            pltpu.VMEM((1,), jnp.int32),
            pltpu.VMEM((_LANES,), jnp.float32),
            pltpu.VMEM((1, n_units), jnp.float32),
            pltpu.VMEM((slc,), jnp.float32),
            pltpu.SemaphoreType.DMA,
            pltpu.SemaphoreType.DMA,
            pltpu.SemaphoreType.DMA,
        ],
    )(_hat_mask_body)
    return f(embeddings, idx, s_vec)


# speedup vs baseline: 1.1979x; 1.0591x over previous
import functools

import jax
import jax.numpy as jnp
from jax import lax
from jax.experimental import pallas as pl
from jax.experimental.pallas import tpu as pltpu
from jax.experimental.pallas import tpu_sc as plsc

_LANES = 16
_NW = 16


def _hat_mask_body(emb_hbm, idx_hbm, s_hbm, out_hbm, idx_v, s_v, row_v,
                   out_v, sem_i, sem_s, sem_g):
    slc = out_v.shape[0]
    wid = lax.axis_index("s")
    cp_i = pltpu.async_copy(idx_hbm, idx_v, sem_i)
    cp_s = pltpu.async_copy(s_hbm, s_v, sem_s)
    cp_i.wait()
    base = wid * slc
    cp_g = pltpu.async_copy(emb_hbm.at[idx_v, pl.ds(base, slc)], row_v, sem_g)
    cp_s.wait()
    cp_g.wait()
    sv = s_v[...]
    for j in range(slc // _LANES):
        x = row_v[0, pl.ds(j * _LANES, _LANES)]
        out_v[pl.ds(j * _LANES, _LANES)] = 1.0 / (1.0 + jnp.exp(-(sv * x)))
    pltpu.sync_copy(out_v, out_hbm.at[pl.ds(base, slc)])


def kernel(embeddings, task_id, s):
    n_tasks, n_units = embeddings.shape
    slc = n_units // _NW
    idx = jnp.reshape(jnp.int32(task_id), (1,))
    s_vec = jnp.full((_LANES,), s, dtype=jnp.float32)

    f = functools.partial(
        pl.kernel,
        out_type=jax.ShapeDtypeStruct((n_units,), jnp.float32),
        mesh=plsc.VectorSubcoreMesh(core_axis_name="c", subcore_axis_name="s",
                                    num_cores=1),
        compiler_params=pltpu.CompilerParams(use_tc_tiling_on_sc=True),
        scratch_types=[
            pltpu.VMEM((1,), jnp.int32),
            pltpu.VMEM((_LANES,), jnp.float32),
            pltpu.VMEM((1, slc), jnp.float32),
            pltpu.VMEM((slc,), jnp.float32),
            pltpu.SemaphoreType.DMA,
            pltpu.SemaphoreType.DMA,
            pltpu.SemaphoreType.DMA,
        ],
    )(_hat_mask_body)
    return f(embeddings, idx, s_vec)
